# baseline (speedup 1.0000x reference)
import functools

import jax
import jax.numpy as jnp
from jax import lax
from jax.experimental import pallas as pl
from jax.experimental.pallas import tpu as pltpu
from jax.experimental.pallas import tpu_sc as plsc

N_NODES = 100000
N_NODES_PAD = 102400
N_EDGES = 6400000
NUM_WORKERS = 32
EDGES_PER_TILE = N_EDGES // NUM_WORKERS
CHUNK = 4000
N_CHUNKS = EDGES_PER_TILE // CHUNK
N_PAIRS = N_CHUNKS // 2


def _sigmoid_body(x_ref, o_ref):
    o_ref[...] = 1.0 / (1.0 + jnp.exp(-x_ref[...]))


def _node_sigmoid(logits):
    x = jnp.pad(logits.reshape(-1), (0, N_NODES_PAD - N_NODES))
    y = pl.pallas_call(
        _sigmoid_body,
        out_shape=jax.ShapeDtypeStruct((N_NODES_PAD // 128, 128), jnp.float32),
    )(x.reshape(N_NODES_PAD // 128, 128))
    return y.reshape(-1)


_mesh = plsc.VectorSubcoreMesh(core_axis_name="c", subcore_axis_name="s")


@functools.partial(
    pl.kernel,
    mesh=_mesh,
    compiler_params=pltpu.CompilerParams(
        use_tc_tiling_on_sc=False, needs_layout_passes=False
    ),
    out_type=jax.ShapeDtypeStruct((N_EDGES,), jnp.float32),
    scratch_types=[
        pltpu.VMEM((N_NODES_PAD,), jnp.float32),
        pltpu.VMEM((2, 2, CHUNK), jnp.int32),
        pltpu.VMEM((2, CHUNK), jnp.float32),
        pltpu.SemaphoreType.DMA,
        pltpu.SemaphoreType.DMA,
        pltpu.SemaphoreType.DMA,
        pltpu.SemaphoreType.DMA,
    ],
)
def _edge_attn_sc(
    table_hbm, ei_hbm, out_hbm,
    table_v, pair_v, out_v,
    sin0, sin1, sout0, sout1,
):
    wid = lax.axis_index("s") * 2 + lax.axis_index("c")
    base = wid * EDGES_PER_TILE
    sin = (sin0, sin1)
    sout = (sout0, sout1)

    def start_in(c, b):
        off = base + c * CHUNK
        pltpu.async_copy(ei_hbm.at[:, pl.ds(off, CHUNK)], pair_v.at[b], sin[b])

    def wait_in(b):
        pltpu.make_async_copy(
            ei_hbm.at[:, pl.ds(0, CHUNK)], pair_v.at[b], sin[b]
        ).wait()

    def start_out(c, b):
        off = base + c * CHUNK
        pltpu.async_copy(out_v.at[b], out_hbm.at[pl.ds(off, CHUNK)], sout[b])

    def wait_out(b):
        pltpu.make_async_copy(
            out_v.at[b], out_hbm.at[pl.ds(0, CHUNK)], sout[b]
        ).wait()

    start_in(0, 0)
    start_in(1, 1)
    pltpu.sync_copy(table_hbm, table_v)

    def pair_body(p, carry):
        for b in range(2):
            c = p * 2 + b
            wait_in(b)

            @pl.when(p > 0)
            def _():
                wait_out(b)

            sv = pair_v.at[b, 0]
            dv = pair_v.at[b, 1]
            ov = out_v.at[b]

            @plsc.parallel_loop(0, CHUNK, step=16, unroll=4)
            def _(i):
                s_idx = sv[pl.ds(i, 16)]
                d_idx = dv[pl.ds(i, 16)]
                vs = plsc.load_gather(table_v, [s_idx])
                vd = plsc.load_gather(table_v, [d_idx])
                ov[pl.ds(i, 16)] = vs * vd

            start_out(c, b)

            @pl.when(p < N_PAIRS - 1)
            def _():
                start_in(c + 2, b)

        return carry

    lax.fori_loop(0, N_PAIRS, pair_body, 0)
    wait_out(0)
    wait_out(1)


def kernel(node_attn_log_logits, edge_index):
    table = _node_sigmoid(node_attn_log_logits)
    out = _edge_attn_sc(table, edge_index.astype(jnp.int32))
    return out.reshape(N_EDGES, 1)


# speedup vs baseline: 1.0039x; 1.0039x over previous
import functools

import jax
import jax.numpy as jnp
from jax import lax
from jax.experimental import pallas as pl
from jax.experimental.pallas import tpu as pltpu
from jax.experimental.pallas import tpu_sc as plsc

N_NODES = 100000
N_NODES_PAD = 102400
N_EDGES = 6400000
NUM_WORKERS = 32
EDGES_PER_TILE = N_EDGES // NUM_WORKERS
CHUNK = 4000
N_CHUNKS = EDGES_PER_TILE // CHUNK
N_PAIRS = N_CHUNKS // 2


def _sigmoid_body(x_ref, o_ref):
    o_ref[...] = 1.0 / (1.0 + jnp.exp(-x_ref[...]))


def _node_sigmoid(logits):
    x = jnp.pad(logits.reshape(-1), (0, N_NODES_PAD - N_NODES))
    y = pl.pallas_call(
        _sigmoid_body,
        out_shape=jax.ShapeDtypeStruct((N_NODES_PAD // 128, 128), jnp.float32),
    )(x.reshape(N_NODES_PAD // 128, 128))
    return y.reshape(-1)


_mesh = plsc.VectorSubcoreMesh(core_axis_name="c", subcore_axis_name="s")


@functools.partial(
    pl.kernel,
    mesh=_mesh,
    compiler_params=pltpu.CompilerParams(
        use_tc_tiling_on_sc=False, needs_layout_passes=False
    ),
    out_type=jax.ShapeDtypeStruct((N_EDGES,), jnp.float32),
    scratch_types=[
        pltpu.VMEM((N_NODES_PAD,), jnp.float32),
        pltpu.VMEM((2, 2, CHUNK), jnp.int32),
        pltpu.VMEM((2, CHUNK), jnp.float32),
        pltpu.SemaphoreType.DMA,
        pltpu.SemaphoreType.DMA,
        pltpu.SemaphoreType.DMA,
        pltpu.SemaphoreType.DMA,
    ],
)
def _edge_attn_sc(
    table_hbm, ei_hbm, out_hbm,
    table_v, pair_v, out_v,
    sin0, sin1, sout0, sout1,
):
    wid = lax.axis_index("s") * 2 + lax.axis_index("c")
    base = wid * EDGES_PER_TILE
    sin = (sin0, sin1)
    sout = (sout0, sout1)

    def start_in(c, b):
        off = base + c * CHUNK
        pltpu.async_copy(ei_hbm.at[:, pl.ds(off, CHUNK)], pair_v.at[b], sin[b])

    def wait_in(b):
        pltpu.make_async_copy(
            ei_hbm.at[:, pl.ds(0, CHUNK)], pair_v.at[b], sin[b]
        ).wait()

    def start_out(c, b):
        off = base + c * CHUNK
        pltpu.async_copy(out_v.at[b], out_hbm.at[pl.ds(off, CHUNK)], sout[b])

    def wait_out(b):
        pltpu.make_async_copy(
            out_v.at[b], out_hbm.at[pl.ds(0, CHUNK)], sout[b]
        ).wait()

    start_in(0, 0)
    start_in(1, 1)
    pltpu.sync_copy(table_hbm, table_v)

    def pair_body(p, carry):
        for b in range(2):
            c = p * 2 + b
            wait_in(b)

            @pl.when(p > 0)
            def _():
                wait_out(b)

            sv = pair_v.at[b, 0]
            dv = pair_v.at[b, 1]
            ov = out_v.at[b]

            @plsc.parallel_loop(0, CHUNK, step=16, unroll=8)
            def _(i):
                s_idx = sv[pl.ds(i, 16)]
                d_idx = dv[pl.ds(i, 16)]
                vs = plsc.load_gather(table_v, [s_idx])
                vd = plsc.load_gather(table_v, [d_idx])
                ov[pl.ds(i, 16)] = vs * vd

            start_out(c, b)

            @pl.when(p < N_PAIRS - 1)
            def _():
                start_in(c + 2, b)

        return carry

    lax.fori_loop(0, N_PAIRS, pair_body, 0)
    wait_out(0)
    wait_out(1)


def kernel(node_attn_log_logits, edge_index):
    table = _node_sigmoid(node_attn_log_logits)
    out = _edge_attn_sc(table, edge_index.astype(jnp.int32))
    return out.reshape(N_EDGES, 1)
